# Initial kernel scaffold; baseline (speedup 1.0000x reference)
import jax
import jax.numpy as jnp
import numpy as np
from jax import lax
from jax.experimental import pallas as pl
from jax.experimental.pallas import tpu as pltpu
from jax.experimental.pallas import tpu_sc as plsc

_NUM_LAYERS = 8
_DEPTH = 10
_TREE_DIM = 4
_INPUT_DIM = 256
_BATCH = 16384
_LEAVES = 2 ** _DEPTH
_SEL = _NUM_LAYERS * _DEPTH
_PROWS = _NUM_LAYERS * _LEAVES

_BT = 2048

_M_NP = np.zeros((_SEL, _NUM_LAYERS), np.float32)
for _l in range(_NUM_LAYERS):
    for _d in range(_DEPTH):
        _M_NP[_l * _DEPTH + _d, _l] = float(2 ** (_DEPTH - 1 - _d))


def _tc_body(x_ref, w_ref, b_ref, m_ref, leaves_ref, fcw_ref, fcb_ref,
             gidx_ref, p_ref):
    logits = jax.lax.dot_general(
        x_ref[...], w_ref[...], (((1,), (0,)), ((), ())),
        precision=jax.lax.Precision.HIGHEST,
        preferred_element_type=jnp.float32) + b_ref[...]
    bits = (logits > 0.0).astype(jnp.float32)
    idxf = jnp.dot(bits, m_ref[...], preferred_element_type=jnp.float32)
    offs = lax.broadcasted_iota(jnp.float32, (1, _NUM_LAYERS), 1) * float(_LEAVES)
    gidx_ref[...] = (idxf + offs).astype(jnp.int32)

    @pl.when(pl.program_id(0) == 0)
    def _():
        for l in range(_NUM_LAYERS):
            tbl = jax.lax.dot_general(
                leaves_ref[l], fcw_ref[l * _TREE_DIM:(l + 1) * _TREE_DIM, :],
                (((1,), (0,)), ((), ())),
                precision=jax.lax.Precision.HIGHEST,
                preferred_element_type=jnp.float32)
            p_ref[l * _LEAVES:(l + 1) * _LEAVES, :] = (
                tbl + fcb_ref[...] * (1.0 / _NUM_LAYERS))


def _tc_stage(x, w, b, m, leaves, fcw, fcb):
    return pl.pallas_call(
        _tc_body,
        grid=(_BATCH // _BT,),
        in_specs=[
            pl.BlockSpec((_BT, _INPUT_DIM), lambda i: (i, 0)),
            pl.BlockSpec((_INPUT_DIM, _SEL), lambda i: (0, 0)),
            pl.BlockSpec((1, _SEL), lambda i: (0, 0)),
            pl.BlockSpec((_SEL, _NUM_LAYERS), lambda i: (0, 0)),
            pl.BlockSpec((_NUM_LAYERS, _LEAVES, _TREE_DIM), lambda i: (0, 0, 0)),
            pl.BlockSpec((_NUM_LAYERS * _TREE_DIM, 2), lambda i: (0, 0)),
            pl.BlockSpec((1, 2), lambda i: (0, 0)),
        ],
        out_specs=[
            pl.BlockSpec((_BT, _NUM_LAYERS), lambda i: (i, 0)),
            pl.BlockSpec((_PROWS, 2), lambda i: (0, 0)),
        ],
        out_shape=[
            jax.ShapeDtypeStruct((_BATCH, _NUM_LAYERS), jnp.int32),
            jax.ShapeDtypeStruct((_PROWS, 2), jnp.float32),
        ],
        compiler_params=pltpu.CompilerParams(
            dimension_semantics=("arbitrary",)),
    )(x, w, b, m, leaves, fcw, fcb)


_SC_INFO = plsc.get_sparse_core_info()
_NC = _SC_INFO.num_cores
_NS = _SC_INFO.num_subcores
_NW = _NC * _NS
_BPW = _BATCH // _NW
_GROUPS = _BPW // 16


def _sc_body(gidx_hbm, p_hbm, out_hbm, idx_v, p_v, out_v):
    wid = lax.axis_index("s") * _NC + lax.axis_index("c")
    base = wid * _BPW
    pltpu.sync_copy(p_hbm, p_v)
    pltpu.sync_copy(gidx_hbm.at[pl.ds(base, _BPW)], idx_v)
    iota = lax.iota(jnp.int32, 16)
    zeros = jnp.zeros((16,), jnp.int32)
    ones = jnp.ones((16,), jnp.int32)

    def group(g, carry):
        rows = g * 16 + iota
        acc0 = jnp.zeros((16,), jnp.float32)
        acc1 = jnp.zeros((16,), jnp.float32)
        for l in range(_NUM_LAYERS):
            lane_l = jnp.full((16,), l, jnp.int32)
            iv = plsc.load_gather(idx_v, [rows, lane_l])
            acc0 = acc0 + plsc.load_gather(p_v, [iv, zeros])
            acc1 = acc1 + plsc.load_gather(p_v, [iv, ones])
        plsc.store_scatter(out_v, [rows, zeros], acc0)
        plsc.store_scatter(out_v, [rows, ones], acc1)
        return carry

    lax.fori_loop(0, _GROUPS, group, 0)
    pltpu.sync_copy(out_v, out_hbm.at[pl.ds(base, _BPW)])


_sc_stage = pl.kernel(
    _sc_body,
    out_type=jax.ShapeDtypeStruct((_BATCH, 2), jnp.float32),
    mesh=plsc.VectorSubcoreMesh(core_axis_name="c", subcore_axis_name="s"),
    scratch_types=[
        pltpu.VMEM((_BPW, _NUM_LAYERS), jnp.int32),
        pltpu.VMEM((_PROWS, 2), jnp.float32),
        pltpu.VMEM((_BPW, 2), jnp.float32),
    ],
)


def kernel(x, sel_w, sel_b, leaves, fc_w, fc_b):
    w = jnp.transpose(sel_w.reshape(_SEL, _INPUT_DIM))
    b = sel_b.reshape(1, _SEL)
    fcb = fc_b.reshape(1, 2)
    m = jnp.asarray(_M_NP)
    gidx, p = _tc_stage(x, w, b, m, leaves, fc_w, fcb)
    return _sc_stage(gidx, p)


# speedup vs baseline: 9.8007x; 9.8007x over previous
import functools

import jax
import jax.numpy as jnp
import numpy as np
from jax import lax
from jax.experimental import pallas as pl
from jax.experimental.pallas import tpu as pltpu
from jax.experimental.pallas import tpu_sc as plsc

_NUM_LAYERS = 8
_DEPTH = 10
_TREE_DIM = 4
_INPUT_DIM = 256
_BATCH = 16384
_LEAVES = 2 ** _DEPTH
_SEL = _NUM_LAYERS * _DEPTH
_PROWS = _NUM_LAYERS * _LEAVES

_BT = 2048

_M_NP = np.zeros((_SEL, _NUM_LAYERS), np.float32)
for _l in range(_NUM_LAYERS):
    for _d in range(_DEPTH):
        _M_NP[_l * _DEPTH + _d, _l] = float(2 ** (_DEPTH - 1 - _d))


def _tc_body(x_ref, w_ref, b_ref, m_ref, leaves_ref, fcw_ref, fcb_ref,
             gidx_ref, p_ref):
    logits = jax.lax.dot_general(
        x_ref[...], w_ref[...], (((1,), (0,)), ((), ())),
        preferred_element_type=jnp.float32) + b_ref[...]
    bits = (logits > 0.0).astype(jnp.float32)
    idxf = jnp.dot(bits, m_ref[...], preferred_element_type=jnp.float32)
    offs = lax.broadcasted_iota(jnp.int32, (1, _NUM_LAYERS), 1) * _LEAVES
    gidx_ref[...] = idxf.astype(jnp.int32) + offs

    @pl.when(pl.program_id(0) == 0)
    def _():
        for l in range(_NUM_LAYERS):
            tbl = jax.lax.dot_general(
                leaves_ref[l], fcw_ref[l * _TREE_DIM:(l + 1) * _TREE_DIM, :],
                (((1,), (0,)), ((), ())),
                preferred_element_type=jnp.float32)
            p_ref[l * _LEAVES:(l + 1) * _LEAVES, :] = (
                tbl + fcb_ref[...] * (1.0 / _NUM_LAYERS))


def _tc_stage(x, w, b, m, leaves, fcw, fcb):
    return pl.pallas_call(
        _tc_body,
        grid=(_BATCH // _BT,),
        in_specs=[
            pl.BlockSpec((_BT, _INPUT_DIM), lambda i: (i, 0)),
            pl.BlockSpec((_INPUT_DIM, _SEL), lambda i: (0, 0)),

            pl.BlockSpec((1, _SEL), lambda i: (0, 0)),
            pl.BlockSpec((_SEL, _NUM_LAYERS), lambda i: (0, 0)),
            pl.BlockSpec((_NUM_LAYERS, _LEAVES, _TREE_DIM), lambda i: (0, 0, 0)),
            pl.BlockSpec((_NUM_LAYERS * _TREE_DIM, 2), lambda i: (0, 0)),
            pl.BlockSpec((1, 2), lambda i: (0, 0)),
        ],
        out_specs=[
            pl.BlockSpec((_BT, _NUM_LAYERS), lambda i: (i, 0)),
            pl.BlockSpec((_PROWS, 2), lambda i: (0, 0)),
        ],
        out_shape=[
            jax.ShapeDtypeStruct((_BATCH, _NUM_LAYERS), jnp.int32),
            jax.ShapeDtypeStruct((_PROWS, 2), jnp.float32),
        ],
        compiler_params=pltpu.CompilerParams(
            dimension_semantics=("arbitrary",)),
    )(x, w, b, m, leaves, fcw, fcb)


_NC = 2
_NS = 16
_NW = _NC * _NS
_BPW = _BATCH // _NW
_GROUPS = _BPW // 16


def _sc_body(gidx_hbm, p_hbm, out_hbm, idx_v, p_v, out_v):
    wid = lax.axis_index("s") * _NC + lax.axis_index("c")
    base = wid * _BPW
    pltpu.sync_copy(p_hbm, p_v)
    pltpu.sync_copy(gidx_hbm.at[pl.ds(base * _NUM_LAYERS, _BPW * _NUM_LAYERS)],
                    idx_v)
    iota = lax.iota(jnp.int32, 16)

    def group(g, carry):
        acc0 = jnp.zeros((16,), jnp.float32)
        acc1 = jnp.zeros((16,), jnp.float32)
        idx_base = g * (16 * _NUM_LAYERS) + iota * _NUM_LAYERS
        for l in range(_NUM_LAYERS):
            iv = plsc.load_gather(idx_v, [idx_base + l])
            acc0 = acc0 + plsc.load_gather(p_v, [iv * 2])
            acc1 = acc1 + plsc.load_gather(p_v, [iv * 2 + 1])
        out_base = g * 32 + iota * 2
        plsc.store_scatter(out_v, [out_base], acc0)
        plsc.store_scatter(out_v, [out_base + 1], acc1)
        return carry

    lax.fori_loop(0, _GROUPS, group, 0)
    pltpu.sync_copy(out_v, out_hbm.at[pl.ds(base * 2, _BPW * 2)])


@functools.cache
def _sc_stage():
    return pl.kernel(
        _sc_body,
        out_type=jax.ShapeDtypeStruct((_BATCH * 2,), jnp.float32),
        mesh=plsc.VectorSubcoreMesh(core_axis_name="c", subcore_axis_name="s",
                                    num_cores=_NC, num_subcores=_NS),
        compiler_params=pltpu.CompilerParams(needs_layout_passes=False),
        scratch_types=[
            pltpu.VMEM((_BPW * _NUM_LAYERS,), jnp.int32),
            pltpu.VMEM((_PROWS * 2,), jnp.float32),
            pltpu.VMEM((_BPW * 2,), jnp.float32),
        ],
    )


def kernel(x, sel_w, sel_b, leaves, fc_w, fc_b):
    xb = x.astype(jnp.bfloat16)
    w = jnp.transpose(sel_w.reshape(_SEL, _INPUT_DIM)).astype(jnp.bfloat16)
    b = sel_b.reshape(1, _SEL)
    fcb = fc_b.reshape(1, 2)
    m = jnp.asarray(_M_NP)
    gidx, p = _tc_stage(xb, w, b, m, leaves, fc_w, fcb)
    out_flat = _sc_stage()(gidx.reshape(-1), p.reshape(-1))
    return out_flat.reshape(_BATCH, 2)


# speedup vs baseline: 11.1953x; 1.1423x over previous
import functools

import jax
import jax.numpy as jnp
import numpy as np
from jax import lax
from jax.experimental import pallas as pl
from jax.experimental.pallas import tpu as pltpu
from jax.experimental.pallas import tpu_sc as plsc

_NUM_LAYERS = 8
_DEPTH = 10
_TREE_DIM = 4
_INPUT_DIM = 256
_BATCH = 16384
_LEAVES = 2 ** _DEPTH
_SEL = _NUM_LAYERS * _DEPTH
_PROWS = _NUM_LAYERS * _LEAVES

_BT = 4096

_M_NP = np.zeros((_SEL, _NUM_LAYERS), np.float32)
for _l in range(_NUM_LAYERS):
    for _d in range(_DEPTH):
        _M_NP[_l * _DEPTH + _d, _l] = float(2 ** (_DEPTH - 1 - _d))


def _tc_body(x_ref, w_ref, b_ref, m_ref, leaves_ref, fcw_ref, fcb_ref,
             gidx_ref, p_ref):
    logits = jax.lax.dot_general(
        x_ref[...].astype(jnp.bfloat16), w_ref[...], (((1,), (0,)), ((), ())),
        preferred_element_type=jnp.float32) + b_ref[...]
    bits = (logits > 0.0).astype(jnp.float32)
    idxf = jnp.dot(bits, m_ref[...], preferred_element_type=jnp.float32)
    offs = lax.broadcasted_iota(jnp.int32, (1, _NUM_LAYERS), 1) * _LEAVES
    gidx_ref[...] = idxf.astype(jnp.int32) + offs

    @pl.when(pl.program_id(0) == 0)
    def _():
        for l in range(_NUM_LAYERS):
            tbl = jax.lax.dot_general(
                leaves_ref[l], fcw_ref[l * _TREE_DIM:(l + 1) * _TREE_DIM, :],
                (((1,), (0,)), ((), ())),
                preferred_element_type=jnp.float32)
            p_ref[l * _LEAVES:(l + 1) * _LEAVES, :] = (
                tbl + fcb_ref[...] * (1.0 / _NUM_LAYERS))


def _tc_stage(x, w, b, m, leaves, fcw, fcb):
    return pl.pallas_call(
        _tc_body,
        grid=(_BATCH // _BT,),
        in_specs=[
            pl.BlockSpec((_BT, _INPUT_DIM), lambda i: (i, 0)),
            pl.BlockSpec((_INPUT_DIM, _SEL), lambda i: (0, 0)),

            pl.BlockSpec((1, _SEL), lambda i: (0, 0)),
            pl.BlockSpec((_SEL, _NUM_LAYERS), lambda i: (0, 0)),
            pl.BlockSpec((_NUM_LAYERS, _LEAVES, _TREE_DIM), lambda i: (0, 0, 0)),
            pl.BlockSpec((_NUM_LAYERS * _TREE_DIM, 2), lambda i: (0, 0)),
            pl.BlockSpec((1, 2), lambda i: (0, 0)),
        ],
        out_specs=[
            pl.BlockSpec((_BT, _NUM_LAYERS), lambda i: (i, 0)),
            pl.BlockSpec((_PROWS, 2), lambda i: (0, 0)),
        ],
        out_shape=[
            jax.ShapeDtypeStruct((_BATCH, _NUM_LAYERS), jnp.int32),
            jax.ShapeDtypeStruct((_PROWS, 2), jnp.float32),
        ],
        compiler_params=pltpu.CompilerParams(
            dimension_semantics=("arbitrary",)),
    )(x, w, b, m, leaves, fcw, fcb)


_NC = 2
_NS = 16
_NW = _NC * _NS
_BPW = _BATCH // _NW
_GROUPS = _BPW // 16


def _sc_body(gidx_hbm, p_hbm, out_hbm, idx_v, p_v, out_v):
    wid = lax.axis_index("s") * _NC + lax.axis_index("c")
    base = wid * _BPW
    pltpu.sync_copy(p_hbm, p_v)
    pltpu.sync_copy(gidx_hbm.at[pl.ds(base * _NUM_LAYERS, _BPW * _NUM_LAYERS)],
                    idx_v)
    iota = lax.iota(jnp.int32, 16)

    def group(g, carry):
        acc0 = jnp.zeros((16,), jnp.float32)
        acc1 = jnp.zeros((16,), jnp.float32)
        idx_base = g * (16 * _NUM_LAYERS) + iota * _NUM_LAYERS
        for l in range(_NUM_LAYERS):
            iv = plsc.load_gather(idx_v, [idx_base + l])
            acc0 = acc0 + plsc.load_gather(p_v, [iv * 2])
            acc1 = acc1 + plsc.load_gather(p_v, [iv * 2 + 1])
        out_base = g * 32 + iota * 2
        plsc.store_scatter(out_v, [out_base], acc0)
        plsc.store_scatter(out_v, [out_base + 1], acc1)
        return carry

    lax.fori_loop(0, _GROUPS, group, 0)
    pltpu.sync_copy(out_v, out_hbm.at[pl.ds(base * 2, _BPW * 2)])


@functools.cache
def _sc_stage():
    return pl.kernel(
        _sc_body,
        out_type=jax.ShapeDtypeStruct((_BATCH * 2,), jnp.float32),
        mesh=plsc.VectorSubcoreMesh(core_axis_name="c", subcore_axis_name="s",
                                    num_cores=_NC, num_subcores=_NS),
        compiler_params=pltpu.CompilerParams(needs_layout_passes=False),
        scratch_types=[
            pltpu.VMEM((_BPW * _NUM_LAYERS,), jnp.int32),
            pltpu.VMEM((_PROWS * 2,), jnp.float32),
            pltpu.VMEM((_BPW * 2,), jnp.float32),
        ],
    )


def kernel(x, sel_w, sel_b, leaves, fc_w, fc_b):
    w = jnp.transpose(sel_w.reshape(_SEL, _INPUT_DIM)).astype(jnp.bfloat16)
    b = sel_b.reshape(1, _SEL)
    fcb = fc_b.reshape(1, 2)
    m = jnp.asarray(_M_NP)
    gidx, p = _tc_stage(x, w, b, m, leaves, fc_w, fcb)
    out_flat = _sc_stage()(gidx.reshape(-1), p.reshape(-1))
    return out_flat.reshape(_BATCH, 2)


# speedup vs baseline: 16.8054x; 1.5011x over previous
import functools

import jax
import jax.numpy as jnp
import numpy as np
from jax import lax
from jax.experimental import pallas as pl
from jax.experimental.pallas import tpu as pltpu
from jax.experimental.pallas import tpu_sc as plsc

_NUM_LAYERS = 8
_DEPTH = 10
_TREE_DIM = 4
_INPUT_DIM = 256
_BATCH = 16384
_LEAVES = 2 ** _DEPTH
_SEL = _NUM_LAYERS * _DEPTH
_PROWS = _NUM_LAYERS * _LEAVES

_BT = 4096

_MT_NP = np.zeros((_NUM_LAYERS, _SEL), np.float32)
for _l in range(_NUM_LAYERS):
    for _d in range(_DEPTH):
        _MT_NP[_l, _l * _DEPTH + _d] = float(2 ** (_DEPTH - 1 - _d))


def _tc_body(x_ref, w_ref, b_ref, m_ref, leaves_ref, fcw_ref, fcb_ref,
             gidx_ref, p_ref):
    logits_t = jax.lax.dot_general(
        w_ref[...], x_ref[...].astype(jnp.bfloat16), (((1,), (1,)), ((), ())),
        preferred_element_type=jnp.float32) + b_ref[...]
    bits_t = (logits_t > 0.0).astype(jnp.float32)
    idxf_t = jnp.dot(m_ref[...], bits_t, preferred_element_type=jnp.float32)
    offs = lax.broadcasted_iota(jnp.int32, (_NUM_LAYERS, 1), 0) * _LEAVES
    gidx_ref[...] = idxf_t.astype(jnp.int32) + offs

    @pl.when(pl.program_id(0) == 0)
    def _():
        for l in range(_NUM_LAYERS):
            tbl_t = jax.lax.dot_general(
                fcw_ref[l * _TREE_DIM:(l + 1) * _TREE_DIM, :], leaves_ref[l],
                (((0,), (1,)), ((), ())),
                preferred_element_type=jnp.float32)
            p_ref[:, l * _LEAVES:(l + 1) * _LEAVES] = (
                tbl_t + fcb_ref[...] * (1.0 / _NUM_LAYERS))


def _tc_stage(x, w, b, m, leaves, fcw, fcb):
    return pl.pallas_call(
        _tc_body,
        grid=(_BATCH // _BT,),
        in_specs=[
            pl.BlockSpec((_BT, _INPUT_DIM), lambda i: (i, 0)),
            pl.BlockSpec((_SEL, _INPUT_DIM), lambda i: (0, 0)),
            pl.BlockSpec((_SEL, 1), lambda i: (0, 0)),
            pl.BlockSpec((_NUM_LAYERS, _SEL), lambda i: (0, 0)),
            pl.BlockSpec((_NUM_LAYERS, _LEAVES, _TREE_DIM), lambda i: (0, 0, 0)),
            pl.BlockSpec((_NUM_LAYERS * _TREE_DIM, 2), lambda i: (0, 0)),
            pl.BlockSpec((2, 1), lambda i: (0, 0)),
        ],
        out_specs=[
            pl.BlockSpec((_NUM_LAYERS, _BT), lambda i: (0, i)),
            pl.BlockSpec((2, _PROWS), lambda i: (0, 0)),
        ],
        out_shape=[
            jax.ShapeDtypeStruct((_NUM_LAYERS, _BATCH), jnp.int32),
            jax.ShapeDtypeStruct((2, _PROWS), jnp.float32),
        ],
        compiler_params=pltpu.CompilerParams(
            dimension_semantics=("arbitrary",)),
    )(x, w, b, m, leaves, fcw, fcb)


_NC = 2
_NS = 16
_NW = _NC * _NS
_BPW = _BATCH // _NW
_GROUPS = _BPW // 16


def _sc_body(gidx_hbm, p_hbm, out0_hbm, out1_hbm, idx_v, p_v, o0_v, o1_v):
    wid = lax.axis_index("s") * _NC + lax.axis_index("c")
    base = wid * _BPW
    for o in range(2):
        pltpu.sync_copy(p_hbm.at[o], p_v.at[pl.ds(o * _PROWS, _PROWS)])
    for l in range(_NUM_LAYERS):
        pltpu.sync_copy(gidx_hbm.at[l, pl.ds(base, _BPW)],
                        idx_v.at[pl.ds(l * _BPW, _BPW)])

    def group(g, carry):
        gb = g * 16
        acc0 = jnp.zeros((16,), jnp.float32)
        acc1 = jnp.zeros((16,), jnp.float32)
        for l in range(_NUM_LAYERS):
            iv = idx_v[pl.ds(l * _BPW + gb, 16)]
            acc0 = acc0 + plsc.load_gather(p_v, [iv])
            acc1 = acc1 + plsc.load_gather(p_v, [iv + _PROWS])
        o0_v[pl.ds(gb, 16)] = acc0
        o1_v[pl.ds(gb, 16)] = acc1
        return carry

    lax.fori_loop(0, _GROUPS, group, 0)
    pltpu.sync_copy(o0_v, out0_hbm.at[pl.ds(base, _BPW)])
    pltpu.sync_copy(o1_v, out1_hbm.at[pl.ds(base, _BPW)])


@functools.cache
def _sc_stage():
    return pl.kernel(
        _sc_body,
        out_type=(jax.ShapeDtypeStruct((_BATCH,), jnp.float32),
                  jax.ShapeDtypeStruct((_BATCH,), jnp.float32)),
        mesh=plsc.VectorSubcoreMesh(core_axis_name="c", subcore_axis_name="s",
                                    num_cores=_NC, num_subcores=_NS),
        compiler_params=pltpu.CompilerParams(needs_layout_passes=False),
        scratch_types=[
            pltpu.VMEM((_BPW * _NUM_LAYERS,), jnp.int32),
            pltpu.VMEM((_PROWS * 2,), jnp.float32),
            pltpu.VMEM((_BPW,), jnp.float32),
            pltpu.VMEM((_BPW,), jnp.float32),
        ],
    )


def kernel(x, sel_w, sel_b, leaves, fc_w, fc_b):
    w = sel_w.reshape(_SEL, _INPUT_DIM).astype(jnp.bfloat16)
    b = sel_b.reshape(_SEL, 1)
    fcb = fc_b.reshape(2, 1)
    m = jnp.asarray(_MT_NP)
    gidx_t, p_t = _tc_stage(x, w, b, m, leaves, fc_w, fcb)
    out0, out1 = _sc_stage()(gidx_t, p_t)
    return jnp.stack([out0, out1], axis=1)


# speedup vs baseline: 17.9903x; 1.0705x over previous
import functools

import jax
import jax.numpy as jnp
import numpy as np
from jax import lax
from jax.experimental import pallas as pl
from jax.experimental.pallas import tpu as pltpu
from jax.experimental.pallas import tpu_sc as plsc

_NUM_LAYERS = 8
_DEPTH = 10
_TREE_DIM = 4
_INPUT_DIM = 256
_BATCH = 16384
_LEAVES = 2 ** _DEPTH
_SEL = _NUM_LAYERS * _DEPTH
_PROWS = _NUM_LAYERS * _LEAVES

_BT = 2048

_MT_NP = np.zeros((_NUM_LAYERS, _SEL), np.float32)
for _l in range(_NUM_LAYERS):
    for _d in range(_DEPTH):
        _MT_NP[_l, _l * _DEPTH + _d] = float(2 ** (_DEPTH - 1 - _d))


def _tc_body(x_ref, w_ref, b_ref, m_ref, leaves_ref, fcw_ref, fcb_ref,
             gidx_ref, p_ref):
    logits_t = jax.lax.dot_general(
        w_ref[...], x_ref[...].astype(jnp.bfloat16), (((1,), (1,)), ((), ())),
        preferred_element_type=jnp.float32) + b_ref[...]
    bits_t = (logits_t > 0.0).astype(jnp.float32)
    idxf_t = jnp.dot(m_ref[...], bits_t, preferred_element_type=jnp.float32)
    offs = lax.broadcasted_iota(jnp.int32, (_NUM_LAYERS, 1), 0) * _LEAVES
    gidx_ref[...] = idxf_t.astype(jnp.int32) + offs

    @pl.when(pl.program_id(0) == 0)
    def _():
        for l in range(_NUM_LAYERS):
            tbl_t = jax.lax.dot_general(
                fcw_ref[l * _TREE_DIM:(l + 1) * _TREE_DIM, :], leaves_ref[l],
                (((0,), (1,)), ((), ())),
                preferred_element_type=jnp.float32)
            p_ref[:, l * _LEAVES:(l + 1) * _LEAVES] = (
                tbl_t + fcb_ref[...] * (1.0 / _NUM_LAYERS))


def _tc_stage(x, w, b, m, leaves, fcw, fcb):
    return pl.pallas_call(
        _tc_body,
        grid=(_BATCH // _BT,),
        in_specs=[
            pl.BlockSpec((_BT, _INPUT_DIM), lambda i: (i, 0)),
            pl.BlockSpec((_SEL, _INPUT_DIM), lambda i: (0, 0)),
            pl.BlockSpec((_SEL, 1), lambda i: (0, 0)),
            pl.BlockSpec((_NUM_LAYERS, _SEL), lambda i: (0, 0)),
            pl.BlockSpec((_NUM_LAYERS, _LEAVES, _TREE_DIM), lambda i: (0, 0, 0)),
            pl.BlockSpec((_NUM_LAYERS * _TREE_DIM, 2), lambda i: (0, 0)),
            pl.BlockSpec((2, 1), lambda i: (0, 0)),
        ],
        out_specs=[
            pl.BlockSpec((_NUM_LAYERS, _BT), lambda i: (0, i)),
            pl.BlockSpec((2, _PROWS), lambda i: (0, 0)),
        ],
        out_shape=[
            jax.ShapeDtypeStruct((_NUM_LAYERS, _BATCH), jnp.int32),
            jax.ShapeDtypeStruct((2, _PROWS), jnp.float32),
        ],
        compiler_params=pltpu.CompilerParams(
            dimension_semantics=("arbitrary",)),
    )(x, w, b, m, leaves, fcw, fcb)


_NC = 2
_NS = 16
_NW = _NC * _NS
_BPW = _BATCH // _NW
_GROUPS = _BPW // 16


def _sc_body(gidx_hbm, p_hbm, out0_hbm, out1_hbm, idx_v, p_v, o0_v, o1_v, sem):
    wid = lax.axis_index("s") * _NC + lax.axis_index("c")
    base = wid * _BPW
    descs = []
    for o in range(2):
        descs.append(pltpu.async_copy(
            p_hbm.at[o], p_v.at[pl.ds(o * _PROWS, _PROWS)], sem))
    for l in range(_NUM_LAYERS):
        descs.append(pltpu.async_copy(
            gidx_hbm.at[l, pl.ds(base, _BPW)],
            idx_v.at[pl.ds(l * _BPW, _BPW)], sem))
    for d in descs:
        d.wait()

    def group(g, carry):
        gb = g * 16
        acc0 = jnp.zeros((16,), jnp.float32)
        acc1 = jnp.zeros((16,), jnp.float32)
        for l in range(_NUM_LAYERS):
            iv = idx_v[pl.ds(l * _BPW + gb, 16)]
            acc0 = acc0 + plsc.load_gather(p_v, [iv])
            acc1 = acc1 + plsc.load_gather(p_v, [iv + _PROWS])
        o0_v[pl.ds(gb, 16)] = acc0
        o1_v[pl.ds(gb, 16)] = acc1
        return carry

    lax.fori_loop(0, _GROUPS, group, 0)
    d0 = pltpu.async_copy(o0_v, out0_hbm.at[pl.ds(base, _BPW)], sem)
    d1 = pltpu.async_copy(o1_v, out1_hbm.at[pl.ds(base, _BPW)], sem)
    d0.wait()
    d1.wait()


@functools.cache
def _sc_stage():
    return pl.kernel(
        _sc_body,
        out_type=(jax.ShapeDtypeStruct((_BATCH,), jnp.float32),
                  jax.ShapeDtypeStruct((_BATCH,), jnp.float32)),
        mesh=plsc.VectorSubcoreMesh(core_axis_name="c", subcore_axis_name="s",
                                    num_cores=_NC, num_subcores=_NS),
        compiler_params=pltpu.CompilerParams(needs_layout_passes=False),
        scratch_types=[
            pltpu.VMEM((_BPW * _NUM_LAYERS,), jnp.int32),
            pltpu.VMEM((_PROWS * 2,), jnp.float32),
            pltpu.VMEM((_BPW,), jnp.float32),
            pltpu.VMEM((_BPW,), jnp.float32),
            pltpu.SemaphoreType.DMA,
        ],
    )


def kernel(x, sel_w, sel_b, leaves, fc_w, fc_b):
    w = sel_w.reshape(_SEL, _INPUT_DIM).astype(jnp.bfloat16)
    b = sel_b.reshape(_SEL, 1)
    fcb = fc_b.reshape(2, 1)
    m = jnp.asarray(_MT_NP)
    gidx_t, p_t = _tc_stage(x, w, b, m, leaves, fc_w, fcb)
    out0, out1 = _sc_stage()(gidx_t, p_t)
    return jnp.stack([out0, out1], axis=1)


# speedup vs baseline: 18.9690x; 1.0544x over previous
import functools

import jax
import jax.numpy as jnp
import numpy as np
from jax import lax
from jax.experimental import pallas as pl
from jax.experimental.pallas import tpu as pltpu
from jax.experimental.pallas import tpu_sc as plsc

_NUM_LAYERS = 8
_DEPTH = 10
_TREE_DIM = 4
_INPUT_DIM = 256
_BATCH = 16384
_LEAVES = 2 ** _DEPTH
_SEL = _NUM_LAYERS * _DEPTH
_PROWS = _NUM_LAYERS * _LEAVES

_BT = 4096

_MT_NP = np.zeros((_NUM_LAYERS, _SEL), np.float32)
for _l in range(_NUM_LAYERS):
    for _d in range(_DEPTH):
        _MT_NP[_l, _l * _DEPTH + _d] = float(2 ** (_DEPTH - 1 - _d))


def _tc_body(x_ref, w_ref, b_ref, m_ref, leaves_ref, fcw_ref, fcb_ref,
             gidx_ref, p_ref):
    logits_t = jax.lax.dot_general(
        w_ref[...], x_ref[...].astype(jnp.bfloat16), (((1,), (1,)), ((), ())),
        preferred_element_type=jnp.float32) + b_ref[...]
    bits_t = (logits_t > 0.0).astype(jnp.float32)
    idxf_t = jnp.dot(m_ref[...], bits_t, preferred_element_type=jnp.float32)
    offs = lax.broadcasted_iota(jnp.int32, (_NUM_LAYERS, 1), 0) * _LEAVES
    gidx_ref[...] = idxf_t.astype(jnp.int32) + offs

    @pl.when(pl.program_id(0) == 0)
    def _():
        for l in range(_NUM_LAYERS):
            tbl_t = jax.lax.dot_general(
                fcw_ref[l * _TREE_DIM:(l + 1) * _TREE_DIM, :], leaves_ref[l],
                (((0,), (1,)), ((), ())),
                preferred_element_type=jnp.float32)
            p_ref[:, l * _LEAVES:(l + 1) * _LEAVES] = (
                tbl_t + fcb_ref[...] * (1.0 / _NUM_LAYERS))


def _tc_stage(x, w, b, m, leaves, fcw, fcb):
    x, w, b, m, leaves, fcw, fcb = (
        pltpu.with_memory_space_constraint(a, pltpu.MemorySpace.HBM)
        for a in (x, w, b, m, leaves, fcw, fcb))
    return pl.pallas_call(
        _tc_body,
        grid=(_BATCH // _BT,),
        in_specs=[
            pl.BlockSpec((_BT, _INPUT_DIM), lambda i: (i, 0)),
            pl.BlockSpec((_SEL, _INPUT_DIM), lambda i: (0, 0)),
            pl.BlockSpec((_SEL, 1), lambda i: (0, 0)),
            pl.BlockSpec((_NUM_LAYERS, _SEL), lambda i: (0, 0)),
            pl.BlockSpec((_NUM_LAYERS, _LEAVES, _TREE_DIM), lambda i: (0, 0, 0)),
            pl.BlockSpec((_NUM_LAYERS * _TREE_DIM, 2), lambda i: (0, 0)),
            pl.BlockSpec((2, 1), lambda i: (0, 0)),
        ],
        out_specs=[
            pl.BlockSpec((_NUM_LAYERS, _BT), lambda i: (0, i)),
            pl.BlockSpec((2, _PROWS), lambda i: (0, 0)),
        ],
        out_shape=[
            jax.ShapeDtypeStruct((_NUM_LAYERS, _BATCH), jnp.int32),
            jax.ShapeDtypeStruct((2, _PROWS), jnp.float32),
        ],
        compiler_params=pltpu.CompilerParams(
            dimension_semantics=("arbitrary",)),
    )(x, w, b, m, leaves, fcw, fcb)


_NC = 2
_NS = 16
_NW = _NC * _NS
_BPW = _BATCH // _NW
_GROUPS = _BPW // 16


def _sc_body(gidx_hbm, p_hbm, out0_hbm, out1_hbm, idx_v, p_v, o0_v, o1_v, sem):
    wid = lax.axis_index("s") * _NC + lax.axis_index("c")
    base = wid * _BPW
    descs = []
    for o in range(2):
        descs.append(pltpu.async_copy(
            p_hbm.at[o], p_v.at[pl.ds(o * _PROWS, _PROWS)], sem))
    for l in range(_NUM_LAYERS):
        descs.append(pltpu.async_copy(
            gidx_hbm.at[l, pl.ds(base, _BPW)],
            idx_v.at[pl.ds(l * _BPW, _BPW)], sem))
    for d in descs:
        d.wait()

    def group(g, carry):
        gb = g * 16
        acc0 = jnp.zeros((16,), jnp.float32)
        acc1 = jnp.zeros((16,), jnp.float32)
        for l in range(_NUM_LAYERS):
            iv = idx_v[pl.ds(l * _BPW + gb, 16)]
            acc0 = acc0 + plsc.load_gather(p_v, [iv])
            acc1 = acc1 + plsc.load_gather(p_v, [iv + _PROWS])
        o0_v[pl.ds(gb, 16)] = acc0
        o1_v[pl.ds(gb, 16)] = acc1
        return carry

    lax.fori_loop(0, _GROUPS, group, 0)
    d0 = pltpu.async_copy(o0_v, out0_hbm.at[pl.ds(base, _BPW)], sem)
    d1 = pltpu.async_copy(o1_v, out1_hbm.at[pl.ds(base, _BPW)], sem)
    d0.wait()
    d1.wait()


@functools.cache
def _sc_stage():
    return pl.kernel(
        _sc_body,
        out_type=(jax.ShapeDtypeStruct((_BATCH,), jnp.float32),
                  jax.ShapeDtypeStruct((_BATCH,), jnp.float32)),
        mesh=plsc.VectorSubcoreMesh(core_axis_name="c", subcore_axis_name="s",
                                    num_cores=_NC, num_subcores=_NS),
        compiler_params=pltpu.CompilerParams(needs_layout_passes=False),
        scratch_types=[
            pltpu.VMEM((_BPW * _NUM_LAYERS,), jnp.int32),
            pltpu.VMEM((_PROWS * 2,), jnp.float32),
            pltpu.VMEM((_BPW,), jnp.float32),
            pltpu.VMEM((_BPW,), jnp.float32),
            pltpu.SemaphoreType.DMA,
        ],
    )


def kernel(x, sel_w, sel_b, leaves, fc_w, fc_b):
    w = sel_w.reshape(_SEL, _INPUT_DIM).astype(jnp.bfloat16)
    b = sel_b.reshape(_SEL, 1)
    fcb = fc_b.reshape(2, 1)
    m = jnp.asarray(_MT_NP)
    gidx_t, p_t = _tc_stage(x, w, b, m, leaves, fc_w, fcb)
    out0, out1 = _sc_stage()(gidx_t, p_t)
    return jnp.stack([out0, out1], axis=1)
